# Initial kernel scaffold; baseline (speedup 1.0000x reference)
import jax
import jax.numpy as jnp
from jax.experimental import pallas as pl


def kernel(h, t, r, y, ent1_embeddings, ent2_embeddings, ent_embeddings, rel1_embeddings, rel2_embeddings, rel_embeddings):
    raise NotImplementedError("write your pallas kernel here")


# speedup vs baseline: 1.6154x; 1.6154x over previous
import functools

import jax
import jax.numpy as jnp
from jax import lax
from jax.experimental import pallas as pl
from jax.experimental.pallas import tpu as pltpu
from jax.experimental.pallas import tpu_sc as plsc

ENT_TOTAL = 100000
REL_TOTAL = 1000
HIDDEN = 64
HALF = HIDDEN // 2
BATCH = 16384
LMBDA = 0.0001

NC = 2
NS = 16
LANES = 16
NW = NC * NS
ROWS_PER_W = BATCH // NW
CHUNK = 128
NCHUNK = ROWS_PER_W // CHUNK
NBUF = 2


def _row_block(refs, i, acc32, acc64):
    e1h_r, e2h_r, eh_r, e1t_r, e2t_r, et_r, r1_r, r2_r, rel_r = refs
    comp = jnp.zeros((LANES,), jnp.float32)
    dist = jnp.zeros((LANES,), jnp.float32)
    for c in range(0, HALF, LANES):
        a1 = e1h_r[i, pl.ds(c, LANES)]
        a2 = e2h_r[i, pl.ds(c, LANES)]
        b1 = e1t_r[i, pl.ds(c, LANES)]
        b2 = e2t_r[i, pl.ds(c, LANES)]
        q1 = r1_r[i, pl.ds(c, LANES)]
        q2 = r2_r[i, pl.ds(c, LANES)]
        comp = comp + (a1 * b1 + a2 * b2) * q1 + (a1 * b2 - a2 * b1) * q2
        acc32 = acc32 + a1 * a1 + a2 * a2 + b1 * b1 + b2 * b2 + q1 * q1 + q2 * q2
    for c in range(0, HIDDEN, LANES):
        x = eh_r[i, pl.ds(c, LANES)]
        z = et_r[i, pl.ds(c, LANES)]
        w = rel_r[i, pl.ds(c, LANES)]
        dist = dist + x * z * w
        acc64 = acc64 + x * x + z * z + w * w
    total = jnp.sum(comp + dist)
    return total, acc32, acc64


def _sc_body(h_hbm, t_hbm, r_hbm,
             e1_hbm, e2_hbm, e_hbm, q1_hbm, q2_hbm, qr_hbm,
             res_hbm, p32_hbm, p64_hbm,
             slots, res_v, p32_v, p64_v, sems):
    wid = lax.axis_index("s") * NC + lax.axis_index("c")
    base = wid * ROWS_PER_W
    lane = lax.iota(jnp.int32, LANES)

    def fire(g, s):
        hi, ti, ri, bufs = slots[s]
        off = base + g * CHUNK
        pltpu.sync_copy(h_hbm.at[pl.ds(off, CHUNK)], hi)
        pltpu.sync_copy(t_hbm.at[pl.ds(off, CHUNK)], ti)
        pltpu.sync_copy(r_hbm.at[pl.ds(off, CHUNK)], ri)
        tables = (e1_hbm, e2_hbm, e_hbm, e1_hbm, e2_hbm, e_hbm,
                  q1_hbm, q2_hbm, qr_hbm)
        idxs = (hi, hi, hi, ti, ti, ti, ri, ri, ri)
        return [pltpu.async_copy(tab.at[ix], buf, sems[s])
                for tab, ix, buf in zip(tables, idxs, bufs)]

    pending = {0: fire(0, 0)}
    for g in range(NCHUNK):
        s = g % NBUF
        if g + 1 < NCHUNK:
            pending[g + 1] = fire(g + 1, (g + 1) % NBUF)
        for d in pending.pop(g):
            d.wait()
        bufs = slots[s][3]

        def body(i16, carry, _bufs=bufs, _g=g):
            acc32, acc64 = carry
            res_vec = jnp.zeros((LANES,), jnp.float32)
            for k in range(LANES):
                total, acc32, acc64 = _row_block(_bufs, i16 * LANES + k,
                                                 acc32, acc64)
                res_vec = jnp.where(lane == k, total, res_vec)
            res_v[pl.ds(_g * CHUNK + i16 * LANES, LANES)] = res_vec
            return acc32, acc64

        if g == 0:
            carry = (jnp.zeros((LANES,), jnp.float32),
                     jnp.zeros((LANES,), jnp.float32))
        carry = lax.fori_loop(0, CHUNK // LANES, body, carry)

    acc32, acc64 = carry
    p32_v[...] = acc32
    p64_v[...] = acc64
    pltpu.sync_copy(res_v, res_hbm.at[pl.ds(base, ROWS_PER_W)])
    pltpu.sync_copy(p32_v, p32_hbm.at[wid])
    pltpu.sync_copy(p64_v, p64_hbm.at[wid])


def _make_sc_call():
    mesh = plsc.VectorSubcoreMesh(core_axis_name="c", subcore_axis_name="s")
    slot = lambda: (pltpu.VMEM((CHUNK,), jnp.int32),
                    pltpu.VMEM((CHUNK,), jnp.int32),
                    pltpu.VMEM((CHUNK,), jnp.int32),
                    (pltpu.VMEM((CHUNK, HALF), jnp.float32),
                     pltpu.VMEM((CHUNK, HALF), jnp.float32),
                     pltpu.VMEM((CHUNK, HIDDEN), jnp.float32),
                     pltpu.VMEM((CHUNK, HALF), jnp.float32),
                     pltpu.VMEM((CHUNK, HALF), jnp.float32),
                     pltpu.VMEM((CHUNK, HIDDEN), jnp.float32),
                     pltpu.VMEM((CHUNK, HALF), jnp.float32),
                     pltpu.VMEM((CHUNK, HALF), jnp.float32),
                     pltpu.VMEM((CHUNK, HIDDEN), jnp.float32)))
    return pl.kernel(
        _sc_body,
        out_type=(jax.ShapeDtypeStruct((BATCH,), jnp.float32),
                  jax.ShapeDtypeStruct((NW, LANES), jnp.float32),
                  jax.ShapeDtypeStruct((NW, LANES), jnp.float32)),
        mesh=mesh,
        compiler_params=pltpu.CompilerParams(needs_layout_passes=False,
                                             use_tc_tiling_on_sc=False),
        scratch_types=[
            tuple(slot() for _ in range(NBUF)),
            pltpu.VMEM((ROWS_PER_W,), jnp.float32),
            pltpu.VMEM((LANES,), jnp.float32),
            pltpu.VMEM((LANES,), jnp.float32),
            tuple(pltpu.SemaphoreType.DMA for _ in range(NBUF)),
        ],
    )


def _finish_body(res_ref, y_ref, p32_ref, p64_ref, out_ref):
    z = -y_ref[...] * res_ref[...]
    sp = jnp.maximum(z, 0.0) + jnp.log1p(jnp.exp(-jnp.abs(z)))
    loss = jnp.sum(sp) * (1.0 / BATCH)
    regul = (jnp.sum(p32_ref[...]) * (1.0 / (BATCH * HALF))
             + jnp.sum(p64_ref[...]) * (1.0 / (BATCH * HIDDEN)))
    out_ref[0, 0] = loss + LMBDA * regul


def kernel(h, t, r, y, ent1_embeddings, ent2_embeddings, ent_embeddings,
           rel1_embeddings, rel2_embeddings, rel_embeddings):
    sc = _make_sc_call()
    res, p32, p64 = sc(h.astype(jnp.int32), t.astype(jnp.int32),
                       r.astype(jnp.int32),
                       ent1_embeddings, ent2_embeddings, ent_embeddings,
                       rel1_embeddings, rel2_embeddings, rel_embeddings)
    out = pl.pallas_call(
        _finish_body,
        out_shape=jax.ShapeDtypeStruct((1, 1), jnp.float32),
        out_specs=pl.BlockSpec(memory_space=pltpu.SMEM),
    )(res.reshape(128, 128), y.reshape(128, 128),
      p32.reshape(4, 128), p64.reshape(4, 128))
    return out[0, 0]
